# baseline (speedup 1.0000x reference)
import functools

import jax
import jax.numpy as jnp
from jax import lax
from jax.experimental import pallas as pl
from jax.experimental.pallas import tpu as pltpu
from jax.experimental.pallas import tpu_sc as plsc

NC = 2
NS = 16
NW = NC * NS
L = 16


def _make_sc(B, OBS, ACT):
    NBLK = ACT // 8
    NITEMS = 2 * NBLK
    BW = 2048
    NCH = B // BW
    BPW = B // NW

    mesh = plsc.VectorSubcoreMesh(
        core_axis_name="c", subcore_axis_name="s",
        num_cores=NC, num_subcores=NS)

    @functools.partial(
        pl.kernel,
        mesh=mesh,
        compiler_params=pltpu.CompilerParams(
            use_tc_tiling_on_sc=True, needs_layout_passes=False),
        out_type=(
            jax.ShapeDtypeStruct((ACT, B), jnp.float32),
            jax.ShapeDtypeStruct((ACT, B), jnp.float32),
            jax.ShapeDtypeStruct((B,), jnp.float32),
        ),
        scratch_types=[
            pltpu.VMEM((B,), jnp.int32),
            pltpu.VMEM((8 * 8 * OBS,), jnp.float32),
            pltpu.VMEM((8, BW), jnp.float32),
            pltpu.VMEM((8, BW), jnp.float32),
            pltpu.VMEM((8, BW), jnp.float32),
            pltpu.VMEM((8, BW), jnp.float32),
            pltpu.VMEM((8, BW), jnp.float32),
            pltpu.VMEM((OBS,), jnp.float32),
            pltpu.VMEM((BPW,), jnp.float32),
            pltpu.SemaphoreType.DMA,
            pltpu.SemaphoreType.DMA,
            pltpu.SemaphoreType.DMA,
            pltpu.SemaphoreType.DMA,
            pltpu.SemaphoreType.DMA,
        ],
    )
    def sc_gather(x_hbm, v_hbm, tbl_hbm,
                  qT_hbm, pT_hbm, val_hbm,
                  x_v, tbl_v, T0, T1, T2, T3, T4, v_v, val_v,
                  sw0, sw1, sw2, sw3, sw4):
        wid = lax.axis_index("s") * NC + lax.axis_index("c")
        pltpu.sync_copy(x_hbm, x_v)
        pltpu.sync_copy(v_hbm, v_v)

        vbase = wid * BPW

        @plsc.parallel_loop(0, BPW // L, unroll=8)
        def _(u):
            xi = x_v[pl.ds(vbase + u * L, L)]
            val_v[pl.ds(u * L, L)] = plsc.load_gather(v_v, [xi])

        pltpu.sync_copy(val_v, val_hbm.at[pl.ds(vbase, BPW)])

        T = (T0, T1, T2, T3, T4)
        sw = (sw0, sw1, sw2, sw3, sw4)
        NB = len(T)

        def drain(b):
            pltpu.make_async_copy(
                qT_hbm.at[pl.ds(0, 8), pl.ds(0, BW)], T[b], sw[b]).wait()

        def fill(buf, w, tbase):
            b0 = w * BW

            @plsc.parallel_loop(0, BW // L, unroll=2)
            def _(u):
                xi = x_v[pl.ds(b0 + u * L, L)]
                for dl in range(8):
                    buf[dl, pl.ds(u * L, L)] = plsc.load_gather(
                        tbl_v, [xi + (tbase + dl * OBS)])

        def item_body(item, carry):
            tbase = (item - start) * (8 * OBS)
            row = item * 8
            for w in range(NCH):
                b = w % NB
                if w >= NB:
                    drain(b)
                else:
                    @pl.when(item > start)
                    def _():
                        drain(b)
                fill(T[b], w, tbase)

                @pl.when(item < NBLK)
                def _():
                    pltpu.async_copy(
                        T[b],
                        qT_hbm.at[pl.ds(row, 8), pl.ds(w * BW, BW)],
                        sw[b])

                @pl.when(item >= NBLK)
                def _():
                    pltpu.async_copy(
                        T[b],
                        pT_hbm.at[pl.ds(row - ACT, 8), pl.ds(w * BW, BW)],
                        sw[b])
            return carry

        start = (wid * NITEMS) // NW
        end = ((wid + 1) * NITEMS) // NW
        pltpu.sync_copy(tbl_hbm.at[pl.ds(start * 8 * OBS, 8 * 8 * OBS)],
                        tbl_v)
        lax.fori_loop(start, end, item_body, 0)
        for b in range(NB):
            drain(b)

    return sc_gather


def kernel(x, v, q, pi_w):
    B = x.shape[0]
    ACT, OBS = pi_w.shape
    x32 = x.astype(jnp.int32)
    tbl = jnp.concatenate([q.T, pi_w], axis=0).reshape(-1)
    qvalT, piT, val = _make_sc(B, OBS, ACT)(x32, v, tbl)
    return (piT.T, val, qvalT.T)


# speedup vs baseline: 1.0075x; 1.0075x over previous
import functools

import jax
import jax.numpy as jnp
from jax import lax
from jax.experimental import pallas as pl
from jax.experimental.pallas import tpu as pltpu
from jax.experimental.pallas import tpu_sc as plsc

NC = 2
NS = 16
NW = NC * NS
L = 16


def _make_sc(B, OBS, ACT):
    NBLK = ACT // 8
    NITEMS = 2 * NBLK
    BW = 2048
    NCH = B // BW
    BPW = B // NW

    mesh = plsc.VectorSubcoreMesh(
        core_axis_name="c", subcore_axis_name="s",
        num_cores=NC, num_subcores=NS)

    @functools.partial(
        pl.kernel,
        mesh=mesh,
        compiler_params=pltpu.CompilerParams(
            use_tc_tiling_on_sc=True, needs_layout_passes=False),
        out_type=(
            jax.ShapeDtypeStruct((ACT, B), jnp.float32),
            jax.ShapeDtypeStruct((ACT, B), jnp.float32),
            jax.ShapeDtypeStruct((B,), jnp.float32),
        ),
        scratch_types=[
            pltpu.VMEM((B,), jnp.int32),
            pltpu.VMEM((8 * 8 * OBS,), jnp.float32),
            pltpu.VMEM((8, BW), jnp.float32),
            pltpu.VMEM((8, BW), jnp.float32),
            pltpu.VMEM((8, BW), jnp.float32),
            pltpu.VMEM((8, BW), jnp.float32),
            pltpu.VMEM((8, BW), jnp.float32),
            pltpu.VMEM((OBS,), jnp.float32),
            pltpu.VMEM((BPW,), jnp.float32),
            pltpu.SemaphoreType.DMA,
            pltpu.SemaphoreType.DMA,
            pltpu.SemaphoreType.DMA,
            pltpu.SemaphoreType.DMA,
            pltpu.SemaphoreType.DMA,
        ],
    )
    def sc_gather(x_hbm, v_hbm, tbl_hbm,
                  qT_hbm, pT_hbm, val_hbm,
                  x_v, tbl_v, T0, T1, T2, T3, T4, v_v, val_v,
                  sw0, sw1, sw2, sw3, sw4):
        wid = lax.axis_index("s") * NC + lax.axis_index("c")
        pltpu.sync_copy(x_hbm, x_v)
        pltpu.sync_copy(v_hbm, v_v)

        vbase = wid * BPW

        @plsc.parallel_loop(0, BPW // L, unroll=4)
        def _(u):
            xi = x_v[pl.ds(vbase + u * L, L)]
            val_v[pl.ds(u * L, L)] = plsc.load_gather(v_v, [xi])

        pltpu.sync_copy(val_v, val_hbm.at[pl.ds(vbase, BPW)])

        T = (T0, T1, T2, T3, T4)
        sw = (sw0, sw1, sw2, sw3, sw4)
        NB = len(T)

        def drain(b):
            pltpu.make_async_copy(
                qT_hbm.at[pl.ds(0, 8), pl.ds(0, BW)], T[b], sw[b]).wait()

        def fill(buf, w, tbase):
            b0 = w * BW

            @plsc.parallel_loop(0, BW // L, unroll=4)
            def _(u):
                xi = x_v[pl.ds(b0 + u * L, L)]
                for dl in range(8):
                    buf[dl, pl.ds(u * L, L)] = plsc.load_gather(
                        tbl_v, [xi + (tbase + dl * OBS)])

        def item_body(item, carry):
            tbase = (item - start) * (8 * OBS)
            row = item * 8
            for w in range(NCH):
                b = w % NB
                if w >= NB:
                    drain(b)
                else:
                    @pl.when(item > start)
                    def _():
                        drain(b)
                fill(T[b], w, tbase)

                @pl.when(item < NBLK)
                def _():
                    pltpu.async_copy(
                        T[b],
                        qT_hbm.at[pl.ds(row, 8), pl.ds(w * BW, BW)],
                        sw[b])

                @pl.when(item >= NBLK)
                def _():
                    pltpu.async_copy(
                        T[b],
                        pT_hbm.at[pl.ds(row - ACT, 8), pl.ds(w * BW, BW)],
                        sw[b])
            return carry

        start = (wid * NITEMS) // NW
        end = ((wid + 1) * NITEMS) // NW
        pltpu.sync_copy(tbl_hbm.at[pl.ds(start * 8 * OBS, 8 * 8 * OBS)],
                        tbl_v)
        lax.fori_loop(start, end, item_body, 0)
        for b in range(NB):
            drain(b)

    return sc_gather


def kernel(x, v, q, pi_w):
    B = x.shape[0]
    ACT, OBS = pi_w.shape
    x32 = x.astype(jnp.int32)
    tbl = jnp.concatenate([q.T, pi_w], axis=0).reshape(-1)
    qvalT, piT, val = _make_sc(B, OBS, ACT)(x32, v, tbl)
    return (piT.T, val, qvalT.T)


# speedup vs baseline: 1.0248x; 1.0172x over previous
import functools

import jax
import jax.numpy as jnp
from jax import lax
from jax.experimental import pallas as pl
from jax.experimental.pallas import tpu as pltpu
from jax.experimental.pallas import tpu_sc as plsc

NC = 2
NS = 16
NW = NC * NS
L = 16


def _make_sc(B, OBS, ACT):
    NBLK = ACT // 8
    NITEMS = 2 * NBLK
    BW = 2048
    NCH = B // BW
    BPW = B // NW

    mesh = plsc.VectorSubcoreMesh(
        core_axis_name="c", subcore_axis_name="s",
        num_cores=NC, num_subcores=NS)

    @functools.partial(
        pl.kernel,
        mesh=mesh,
        compiler_params=pltpu.CompilerParams(
            use_tc_tiling_on_sc=True, needs_layout_passes=False),
        out_type=(
            jax.ShapeDtypeStruct((ACT, B), jnp.float32),
            jax.ShapeDtypeStruct((ACT, B), jnp.float32),
            jax.ShapeDtypeStruct((B,), jnp.float32),
        ),
        scratch_types=[
            pltpu.VMEM((B,), jnp.int32),
            pltpu.VMEM((8 * 8 * OBS,), jnp.float32),
            pltpu.VMEM((8, BW), jnp.float32),
            pltpu.VMEM((8, BW), jnp.float32),
            pltpu.VMEM((8, BW), jnp.float32),
            pltpu.VMEM((8, BW), jnp.float32),
            pltpu.VMEM((8, BW), jnp.float32),
            pltpu.VMEM((OBS,), jnp.float32),
            pltpu.VMEM((BPW,), jnp.float32),
            pltpu.SemaphoreType.DMA,
            pltpu.SemaphoreType.DMA,
            pltpu.SemaphoreType.DMA,
            pltpu.SemaphoreType.DMA,
            pltpu.SemaphoreType.DMA,
        ],
    )
    def sc_gather(x_hbm, v_hbm, tbl_hbm,
                  qT_hbm, pT_hbm, val_hbm,
                  x_v, tbl_v, T0, T1, T2, T3, T4, v_v, val_v,
                  sw0, sw1, sw2, sw3, sw4):
        wid = lax.axis_index("s") * NC + lax.axis_index("c")
        start = (wid * NITEMS) // NW
        end = ((wid + 1) * NITEMS) // NW
        c1 = pltpu.async_copy(x_hbm, x_v, sw0)
        c2 = pltpu.async_copy(v_hbm, v_v, sw1)
        c3 = pltpu.async_copy(
            tbl_hbm.at[pl.ds(start * 8 * OBS, 8 * 8 * OBS)], tbl_v, sw2)
        c1.wait()
        c2.wait()
        c3.wait()

        vbase = wid * BPW

        @plsc.parallel_loop(0, BPW // L, unroll=4)
        def _(u):
            xi = x_v[pl.ds(vbase + u * L, L)]
            val_v[pl.ds(u * L, L)] = plsc.load_gather(v_v, [xi])

        pltpu.sync_copy(val_v, val_hbm.at[pl.ds(vbase, BPW)])

        T = (T0, T1, T2, T3, T4)
        sw = (sw0, sw1, sw2, sw3, sw4)
        NB = len(T)

        def drain(b):
            pltpu.make_async_copy(
                qT_hbm.at[pl.ds(0, 8), pl.ds(0, BW)], T[b], sw[b]).wait()

        def fill(buf, w, tbase):
            b0 = w * BW

            @plsc.parallel_loop(0, BW // L, unroll=4)
            def _(u):
                xi = x_v[pl.ds(b0 + u * L, L)]
                for dl in range(8):
                    buf[dl, pl.ds(u * L, L)] = plsc.load_gather(
                        tbl_v, [xi + (tbase + dl * OBS)])

        def item_body(item, carry):
            tbase = (item - start) * (8 * OBS)
            row = item * 8
            for w in range(NCH):
                b = w % NB
                if w >= NB:
                    drain(b)
                else:
                    @pl.when(item > start)
                    def _():
                        drain(b)
                fill(T[b], w, tbase)

                @pl.when(item < NBLK)
                def _():
                    pltpu.async_copy(
                        T[b],
                        qT_hbm.at[pl.ds(row, 8), pl.ds(w * BW, BW)],
                        sw[b])

                @pl.when(item >= NBLK)
                def _():
                    pltpu.async_copy(
                        T[b],
                        pT_hbm.at[pl.ds(row - ACT, 8), pl.ds(w * BW, BW)],
                        sw[b])
            return carry

        lax.fori_loop(start, end, item_body, 0)
        for b in range(NB):
            drain(b)

    return sc_gather


def kernel(x, v, q, pi_w):
    B = x.shape[0]
    ACT, OBS = pi_w.shape
    x32 = x.astype(jnp.int32)
    tbl = jnp.concatenate([q.T, pi_w], axis=0).reshape(-1)
    qvalT, piT, val = _make_sc(B, OBS, ACT)(x32, v, tbl)
    return (piT.T, val, qvalT.T)


# speedup vs baseline: 1.0262x; 1.0013x over previous
import functools

import jax
import jax.numpy as jnp
from jax import lax
from jax.experimental import pallas as pl
from jax.experimental.pallas import tpu as pltpu
from jax.experimental.pallas import tpu_sc as plsc

NC = 2
NS = 16
NW = NC * NS
L = 16


def _make_sc(B, OBS, ACT):
    NBLK = ACT // 8
    NITEMS = 2 * NBLK
    BW = 2048
    NCH = B // BW
    BPW = B // NW

    mesh = plsc.VectorSubcoreMesh(
        core_axis_name="c", subcore_axis_name="s",
        num_cores=NC, num_subcores=NS)

    @functools.partial(
        pl.kernel,
        mesh=mesh,
        compiler_params=pltpu.CompilerParams(
            use_tc_tiling_on_sc=True, needs_layout_passes=False),
        out_type=(
            jax.ShapeDtypeStruct((ACT, B), jnp.float32),
            jax.ShapeDtypeStruct((ACT, B), jnp.float32),
            jax.ShapeDtypeStruct((B,), jnp.float32),
        ),
        scratch_types=[
            pltpu.VMEM((B,), jnp.int32),
            pltpu.VMEM((8 * 8 * OBS,), jnp.float32),
            pltpu.VMEM((8, BW), jnp.float32),
            pltpu.VMEM((8, BW), jnp.float32),
            pltpu.VMEM((8, BW), jnp.float32),
            pltpu.VMEM((8, BW), jnp.float32),
            pltpu.VMEM((8, BW), jnp.float32),
            pltpu.VMEM((OBS,), jnp.float32),
            pltpu.VMEM((BPW,), jnp.float32),
            pltpu.SemaphoreType.DMA,
            pltpu.SemaphoreType.DMA,
            pltpu.SemaphoreType.DMA,
            pltpu.SemaphoreType.DMA,
            pltpu.SemaphoreType.DMA,
        ],
    )
    def sc_gather(x_hbm, v_hbm, tbl_hbm,
                  qT_hbm, pT_hbm, val_hbm,
                  x_v, tbl_v, T0, T1, T2, T3, T4, v_v, val_v,
                  sw0, sw1, sw2, sw3, sw4):
        wid = lax.axis_index("s") * NC + lax.axis_index("c")
        start = (wid * NITEMS) // NW
        end = ((wid + 1) * NITEMS) // NW
        c1 = pltpu.async_copy(x_hbm, x_v, sw0)
        c2 = pltpu.async_copy(v_hbm, v_v, sw1)
        c3 = pltpu.async_copy(
            tbl_hbm.at[pl.ds(start * 8 * OBS, 8 * 8 * OBS)], tbl_v, sw2)
        c1.wait()
        c2.wait()
        c3.wait()

        T = (T0, T1, T2, T3, T4)
        sw = (sw0, sw1, sw2, sw3, sw4)
        NB = len(T)

        def drain(b):
            pltpu.make_async_copy(
                qT_hbm.at[pl.ds(0, 8), pl.ds(0, BW)], T[b], sw[b]).wait()

        def fill(buf, w, tbase):
            b0 = w * BW

            @plsc.parallel_loop(0, BW // L, unroll=4)
            def _(u):
                xi = x_v[pl.ds(b0 + u * L, L)]
                for dl in range(8):
                    buf[dl, pl.ds(u * L, L)] = plsc.load_gather(
                        tbl_v, [xi + (tbase + dl * OBS)])

        def item_body(item, carry):
            tbase = (item - start) * (8 * OBS)
            row = item * 8
            for w in range(NCH):
                b = w % NB
                if w >= NB:
                    drain(b)
                else:
                    @pl.when(item > start)
                    def _():
                        drain(b)
                fill(T[b], w, tbase)

                @pl.when(item < NBLK)
                def _():
                    pltpu.async_copy(
                        T[b],
                        qT_hbm.at[pl.ds(row, 8), pl.ds(w * BW, BW)],
                        sw[b])

                @pl.when(item >= NBLK)
                def _():
                    pltpu.async_copy(
                        T[b],
                        pT_hbm.at[pl.ds(row - ACT, 8), pl.ds(w * BW, BW)],
                        sw[b])
            return carry

        lax.fori_loop(start, end, item_body, 0)
        vbase = wid * BPW

        @plsc.parallel_loop(0, BPW // L, unroll=4)
        def _(u):
            xi = x_v[pl.ds(vbase + u * L, L)]
            val_v[pl.ds(u * L, L)] = plsc.load_gather(v_v, [xi])

        pltpu.sync_copy(val_v, val_hbm.at[pl.ds(vbase, BPW)])

        for b in range(NB):
            drain(b)

    return sc_gather


def kernel(x, v, q, pi_w):
    B = x.shape[0]
    ACT, OBS = pi_w.shape
    x32 = x.astype(jnp.int32)
    tbl = jnp.concatenate([q.T, pi_w], axis=0).reshape(-1)
    qvalT, piT, val = _make_sc(B, OBS, ACT)(x32, v, tbl)
    return (piT.T, val, qvalT.T)
